# Initial kernel scaffold; baseline (speedup 1.0000x reference)
import jax
import jax.numpy as jnp
from jax.experimental import pallas as pl


def kernel(edge_index, h, e, snorm_n, W_post, b_post, gamma, beta, Wr0, br0, Wr1, br1, Wr2, br2):
    raise NotImplementedError("write your pallas kernel here")


# speedup vs baseline: 2.8116x; 2.8116x over previous
import functools

import jax
import jax.numpy as jnp
from jax import lax
from jax.experimental import pallas as pl
from jax.experimental.pallas import tpu as pltpu
from jax.experimental.pallas import tpu_sc as plsc

N = 10000
E = 320000
D = 128
HID = 128
OUT = 10
L = 4
AVG_D_LOG = 3.4657

NC = 2
NS = 16
NW = NC * NS
NPW = 320
PADN = NW * NPW
CH = 160
NCHUNK = NPW // CH
EB = 128
EPAD = E + EB
RPLEN = PADN + 16

_NEG = float("-inf")
_POS = float("inf")


def _sc_agg_body(h_hbm, src_hbm, dst_hbm, rp_hbm,
                 sum_hbm, max_hbm, min_hbm,
                 rp_v, idx_v, dst_v, msg_v, acc_s, acc_x, acc_n, sem):
    wid = lax.axis_index("s") * NC + lax.axis_index("c")
    base = wid * NPW
    pltpu.sync_copy(rp_hbm.at[pl.ds(base, NPW + 16)], rp_v)

    for c in range(NCHUNK):
        n0 = c * CH

        def init_body(i, _):
            for r in range(8):
                sl = pl.ds(r * 16, 16)
                acc_s[i, sl] = jnp.zeros((16,), jnp.float32)
                acc_x[i, sl] = jnp.full((16,), _NEG, jnp.float32)
                acc_n[i, sl] = jnp.full((16,), _POS, jnp.float32)
            return 0

        lax.fori_loop(0, CH, init_body, 0)

        e_lo = rp_v[pl.ds(n0, 16)][0]
        e_hi = rp_v[pl.ds(n0 + CH, 16)][0]
        e0 = (e_lo // 8) * 8
        nblk = (e_hi - e0 + (EB - 1)) // EB
        node_base = base + n0

        def blk_body(b, _):
            blk = e0 + b * EB
            pltpu.sync_copy(src_hbm.at[pl.ds(blk, EB)], idx_v)
            pltpu.sync_copy(dst_hbm.at[pl.ds(blk, EB)], dst_v.at[pl.ds(0, EB)])
            pltpu.async_copy(h_hbm.at[idx_v], msg_v, sem).wait()

            def edge_body(j, _):
                e_abs = blk + j

                @pl.when(jnp.logical_and(e_abs >= e_lo, e_abs < e_hi))
                def _():
                    nl = dst_v[pl.ds(j, 16)][0] - node_base
                    for r in range(8):
                        sl = pl.ds(r * 16, 16)
                        m = msg_v[j, sl]
                        plsc.addupdate(acc_s.at[nl, sl], m)
                        acc_x[nl, sl] = jnp.maximum(acc_x[nl, sl], m)
                        acc_n[nl, sl] = jnp.minimum(acc_n[nl, sl], m)
                return 0

            lax.fori_loop(0, EB, edge_body, 0)
            return 0

        lax.fori_loop(0, nblk, blk_body, 0)

        row0 = base + n0
        pltpu.sync_copy(acc_s, sum_hbm.at[pl.ds(row0, CH), :])
        pltpu.sync_copy(acc_x, max_hbm.at[pl.ds(row0, CH), :])
        pltpu.sync_copy(acc_n, min_hbm.at[pl.ds(row0, CH), :])


def _make_sc_agg():
    mesh = plsc.VectorSubcoreMesh(core_axis_name="c", subcore_axis_name="s")
    return pl.kernel(
        _sc_agg_body,
        out_type=[jax.ShapeDtypeStruct((PADN, D), jnp.float32)] * 3,
        mesh=mesh,
        scratch_types=[
            pltpu.VMEM((NPW + 16,), jnp.int32),
            pltpu.VMEM((EB,), jnp.int32),
            pltpu.VMEM((EB + 16,), jnp.int32),
            pltpu.VMEM((EB, D), jnp.float32),
            pltpu.VMEM((CH, D), jnp.float32),
            pltpu.VMEM((CH, D), jnp.float32),
            pltpu.VMEM((CH, D), jnp.float32),
            pltpu.SemaphoreType.DMA,
        ],
    )


RB = 400
NB = N // RB


def _layer_a_body(h_ref, s_ref, x_ref, n_ref, scal_ref, w_ref, b_ref,
                  p_ref, stats_ref):
    i = pl.program_id(0)
    inv_deg = scal_ref[:, 0:1]
    s_amp = scal_ref[:, 1:2]
    s_att = scal_ref[:, 2:3]
    snorm = scal_ref[:, 3:4]
    mean = s_ref[...] * inv_deg
    mx = x_ref[...]
    mx = jnp.where(jnp.isfinite(mx), mx, 0.0)
    mn = n_ref[...]
    mn = jnp.where(jnp.isfinite(mn), mn, 0.0)
    h = h_ref[...]
    hcat = jnp.concatenate(
        [h, mean, mx, mn,
         mean * s_amp, mx * s_amp, mn * s_amp,
         mean * s_att, mx * s_att, mn * s_att], axis=1)
    p = jnp.dot(hcat, w_ref[...], preferred_element_type=jnp.float32)
    p = p + b_ref[0:1, :]
    p = jnp.maximum(p, 0.0) * snorm
    p_ref[...] = p

    @pl.when(i == 0)
    def _():
        stats_ref[...] = jnp.zeros_like(stats_ref)

    stats_ref[0:1, :] += jnp.sum(p, axis=0, keepdims=True)
    stats_ref[1:2, :] += jnp.sum(p * p, axis=0, keepdims=True)


def _layer_c_body(h_ref, p_ref, ab_ref, o_ref):
    o_ref[...] = h_ref[...] + p_ref[...] * ab_ref[0:1, :] + ab_ref[1:2, :]


def _readout_body(h_ref, w0_ref, b0_ref, w1_ref, b1_ref, w2_ref, b2_ref,
                  o_ref, acc_ref):
    i = pl.program_id(0)

    @pl.when(i == 0)
    def _():
        acc_ref[...] = jnp.zeros_like(acc_ref)

    acc_ref[0:1, :] += jnp.sum(h_ref[...], axis=0, keepdims=True)

    @pl.when(i == NB - 1)
    def _():
        hg = acc_ref[0:1, :] * (1.0 / N)
        x = jnp.maximum(jnp.dot(hg, w0_ref[...],
                                preferred_element_type=jnp.float32)
                        + b0_ref[0:1, :], 0.0)
        x = jnp.maximum(jnp.dot(x, w1_ref[...],
                                preferred_element_type=jnp.float32)
                        + b1_ref[0:1, :], 0.0)
        x = jnp.dot(x, w2_ref[...], preferred_element_type=jnp.float32) \
            + b2_ref[0:1, :]
        o_ref[...] = jnp.broadcast_to(x, o_ref.shape)


def _full_spec(shape):
    return pl.BlockSpec(shape, lambda i: (0,) * len(shape))


_layer_a = pl.pallas_call(
    _layer_a_body,
    grid=(NB,),
    in_specs=[
        pl.BlockSpec((RB, D), lambda i: (i, 0)),
        pl.BlockSpec((RB, D), lambda i: (i, 0)),
        pl.BlockSpec((RB, D), lambda i: (i, 0)),
        pl.BlockSpec((RB, D), lambda i: (i, 0)),
        pl.BlockSpec((RB, 8), lambda i: (i, 0)),
        _full_spec((10 * D, HID)),
        _full_spec((1, HID)),
    ],
    out_specs=[
        pl.BlockSpec((RB, HID), lambda i: (i, 0)),
        _full_spec((8, HID)),
    ],
    out_shape=[
        jax.ShapeDtypeStruct((N, HID), jnp.float32),
        jax.ShapeDtypeStruct((8, HID), jnp.float32),
    ],
)

_layer_c = pl.pallas_call(
    _layer_c_body,
    grid=(NB,),
    in_specs=[
        pl.BlockSpec((RB, D), lambda i: (i, 0)),
        pl.BlockSpec((RB, HID), lambda i: (i, 0)),
        _full_spec((8, HID)),
    ],
    out_specs=pl.BlockSpec((RB, D), lambda i: (i, 0)),
    out_shape=jax.ShapeDtypeStruct((N, D), jnp.float32),
)

_readout = pl.pallas_call(
    _readout_body,
    grid=(NB,),
    in_specs=[
        pl.BlockSpec((RB, D), lambda i: (i, 0)),
        _full_spec((D, HID // 2)),
        _full_spec((1, HID // 2)),
        _full_spec((HID // 2, HID // 4)),
        _full_spec((1, HID // 4)),
        _full_spec((HID // 4, 16)),
        _full_spec((1, 16)),
    ],
    out_specs=_full_spec((8, 16)),
    out_shape=jax.ShapeDtypeStruct((8, 16), jnp.float32),
    scratch_shapes=[pltpu.VMEM((8, HID), jnp.float32)],
)


def kernel(edge_index, h, e, snorm_n, W_post, b_post, gamma, beta,
           Wr0, br0, Wr1, br1, Wr2, br2):
    src = edge_index[0].astype(jnp.int32)
    dst = edge_index[1].astype(jnp.int32)

    order = jnp.argsort(dst)
    dst_s = dst[order]
    src_s = src[order]
    row_ptr = jnp.searchsorted(dst_s, jnp.arange(RPLEN, dtype=jnp.int32),
                               side="left").astype(jnp.int32)
    src_p = jnp.concatenate([src_s, jnp.zeros((EPAD - E,), jnp.int32)])
    dst_p = jnp.concatenate([dst_s, jnp.zeros((EPAD - E,), jnp.int32)])

    deg_raw = (row_ptr[1:N + 1] - row_ptr[:N]).astype(jnp.float32)
    deg = jnp.maximum(deg_raw, 1.0)
    log_deg = jnp.log(deg + 1.0)
    scal = jnp.zeros((N, 8), jnp.float32)
    scal = scal.at[:, 0].set(1.0 / deg)
    scal = scal.at[:, 1].set(log_deg / AVG_D_LOG)
    scal = scal.at[:, 2].set(AVG_D_LOG / log_deg)
    scal = scal.at[:, 3].set(snorm_n[:, 0])

    sc_agg = _make_sc_agg()
    b2d = b_post.reshape(L, 1, HID)

    for l in range(L):
        s_a, x_a, n_a = sc_agg(h, src_p, dst_p, row_ptr)
        p, stats = _layer_a(h, s_a[:N], x_a[:N], n_a[:N], scal,
                            W_post[l], b2d[l])
        mu = stats[0] * (1.0 / N)
        var = stats[1] * (1.0 / N) - mu * mu
        a = gamma[l] / jnp.sqrt(var + 1e-5)
        bb = beta[l] - mu * a
        ab = jnp.concatenate([a.reshape(1, HID), bb.reshape(1, HID),
                              jnp.zeros((6, HID), jnp.float32)], axis=0)
        h = _layer_c(h, p, ab)

    w2p = jnp.zeros((HID // 4, 16), jnp.float32).at[:, :OUT].set(Wr2)
    b2p = jnp.zeros((1, 16), jnp.float32).at[0, :OUT].set(br2)
    outp = _readout(h, Wr0, br0.reshape(1, -1), Wr1, br1.reshape(1, -1),
                    w2p, b2p)
    return outp[0, :OUT]


# speedup vs baseline: 3.4800x; 1.2377x over previous
import functools

import jax
import jax.numpy as jnp
from jax import lax
from jax.experimental import pallas as pl
from jax.experimental.pallas import tpu as pltpu
from jax.experimental.pallas import tpu_sc as plsc

N = 10000
E = 320000
D = 128
HID = 128
OUT = 10
L = 4
AVG_D_LOG = 3.4657

NC = 2
NS = 16
NW = NC * NS
NPW = 320
PADN = NW * NPW
CH = 160
NCHUNK = NPW // CH
EB = 128
EPAD = E + EB
RPLEN = PADN + 16

_NEG = float("-inf")
_POS = float("inf")


def _sc_agg_body(h_hbm, src_hbm, dst_hbm, rp_hbm,
                 sum_hbm, max_hbm, min_hbm,
                 rp_v, idx_v, dst_v, msg_v, acc_s, acc_x, acc_n, sem):
    wid = lax.axis_index("s") * NC + lax.axis_index("c")
    base = wid * NPW
    pltpu.sync_copy(rp_hbm.at[pl.ds(base, NPW + 16)], rp_v)

    for c in range(NCHUNK):
        n0 = c * CH
        cabs = base + n0

        def init_body(i, _):
            for r in range(8):
                sl = pl.ds(r * 16, 16)
                acc_s[i, sl] = jnp.zeros((16,), jnp.float32)
                acc_x[i, sl] = jnp.full((16,), _NEG, jnp.float32)
                acc_n[i, sl] = jnp.full((16,), _POS, jnp.float32)
            return 0

        lax.fori_loop(0, CH, init_body, 0)

        e_lo = rp_v[pl.ds(n0, 16)][0]
        e_hi = rp_v[pl.ds(n0 + CH, 16)][0]
        klo = e_lo // EB
        khi = (e_hi + EB - 1) // EB

        def blk_body(kk, _):
            blk = pl.multiple_of((klo + kk) * EB, 8)
            pltpu.sync_copy(src_hbm.at[pl.ds(blk, EB)], idx_v)
            pltpu.sync_copy(dst_hbm.at[pl.ds(blk, EB)], dst_v)
            pltpu.async_copy(h_hbm.at[idx_v], msg_v, sem).wait()

            first_n = dst_v[pl.ds(0, 16)][0]
            last_n = dst_v[pl.ds(EB - 16, 16)][15]
            lo_n = jnp.maximum(first_n, cabs)
            hi_n = jnp.minimum(last_n + 1, cabs + CH)

            def run_body(rn, _):
                node = lo_n + rn
                ln = node - cabs
                v = rp_v[pl.ds(node - base, 16)]
                le = jnp.maximum(v[0], blk) - blk
                he = jnp.maximum(jnp.minimum(v[1], blk + EB) - blk, le)

                def edge_body(jj, _):
                    j = le + jj
                    for r in range(8):
                        sl = pl.ds(r * 16, 16)
                        m = msg_v[j, sl]
                        plsc.addupdate(acc_s.at[ln, sl], m)
                        acc_x[ln, sl] = jnp.maximum(acc_x[ln, sl], m)
                        acc_n[ln, sl] = jnp.minimum(acc_n[ln, sl], m)
                    return 0

                lax.fori_loop(0, he - le, edge_body, 0)
                return 0

            lax.fori_loop(0, hi_n - lo_n, run_body, 0)
            return 0

        lax.fori_loop(0, khi - klo, blk_body, 0)

        row0 = cabs
        pltpu.sync_copy(acc_s, sum_hbm.at[pl.ds(row0, CH), :])
        pltpu.sync_copy(acc_x, max_hbm.at[pl.ds(row0, CH), :])
        pltpu.sync_copy(acc_n, min_hbm.at[pl.ds(row0, CH), :])


def _make_sc_agg():
    mesh = plsc.VectorSubcoreMesh(core_axis_name="c", subcore_axis_name="s")
    return pl.kernel(
        _sc_agg_body,
        out_type=[jax.ShapeDtypeStruct((PADN, D), jnp.float32)] * 3,
        mesh=mesh,
        scratch_types=[
            pltpu.VMEM((NPW + 16,), jnp.int32),
            pltpu.VMEM((EB,), jnp.int32),
            pltpu.VMEM((EB,), jnp.int32),
            pltpu.VMEM((EB, D), jnp.float32),
            pltpu.VMEM((CH, D), jnp.float32),
            pltpu.VMEM((CH, D), jnp.float32),
            pltpu.VMEM((CH, D), jnp.float32),
            pltpu.SemaphoreType.DMA,
        ],
    )


RB = 400
NB = N // RB


def _layer_a_body(h_ref, s_ref, x_ref, n_ref, scal_ref, w_ref, b_ref,
                  p_ref, stats_ref):
    i = pl.program_id(0)
    inv_deg = scal_ref[:, 0:1]
    s_amp = scal_ref[:, 1:2]
    s_att = scal_ref[:, 2:3]
    snorm = scal_ref[:, 3:4]
    mean = s_ref[...] * inv_deg
    mx = x_ref[...]
    mx = jnp.where(jnp.isfinite(mx), mx, 0.0)
    mn = n_ref[...]
    mn = jnp.where(jnp.isfinite(mn), mn, 0.0)
    h = h_ref[...]
    hcat = jnp.concatenate(
        [h, mean, mx, mn,
         mean * s_amp, mx * s_amp, mn * s_amp,
         mean * s_att, mx * s_att, mn * s_att], axis=1)
    p = jnp.dot(hcat, w_ref[...], preferred_element_type=jnp.float32)
    p = p + b_ref[0:1, :]
    p = jnp.maximum(p, 0.0) * snorm
    p_ref[...] = p

    @pl.when(i == 0)
    def _():
        stats_ref[...] = jnp.zeros_like(stats_ref)

    stats_ref[0:1, :] += jnp.sum(p, axis=0, keepdims=True)
    stats_ref[1:2, :] += jnp.sum(p * p, axis=0, keepdims=True)


def _layer_c_body(h_ref, p_ref, ab_ref, o_ref):
    o_ref[...] = h_ref[...] + p_ref[...] * ab_ref[0:1, :] + ab_ref[1:2, :]


def _readout_body(h_ref, w0_ref, b0_ref, w1_ref, b1_ref, w2_ref, b2_ref,
                  o_ref, acc_ref):
    i = pl.program_id(0)

    @pl.when(i == 0)
    def _():
        acc_ref[...] = jnp.zeros_like(acc_ref)

    acc_ref[0:1, :] += jnp.sum(h_ref[...], axis=0, keepdims=True)

    @pl.when(i == NB - 1)
    def _():
        hg = acc_ref[0:1, :] * (1.0 / N)
        x = jnp.maximum(jnp.dot(hg, w0_ref[...],
                                preferred_element_type=jnp.float32)
                        + b0_ref[0:1, :], 0.0)
        x = jnp.maximum(jnp.dot(x, w1_ref[...],
                                preferred_element_type=jnp.float32)
                        + b1_ref[0:1, :], 0.0)
        x = jnp.dot(x, w2_ref[...], preferred_element_type=jnp.float32) \
            + b2_ref[0:1, :]
        o_ref[...] = jnp.broadcast_to(x, o_ref.shape)


def _full_spec(shape):
    return pl.BlockSpec(shape, lambda i: (0,) * len(shape))


_layer_a = pl.pallas_call(
    _layer_a_body,
    grid=(NB,),
    in_specs=[
        pl.BlockSpec((RB, D), lambda i: (i, 0)),
        pl.BlockSpec((RB, D), lambda i: (i, 0)),
        pl.BlockSpec((RB, D), lambda i: (i, 0)),
        pl.BlockSpec((RB, D), lambda i: (i, 0)),
        pl.BlockSpec((RB, 8), lambda i: (i, 0)),
        _full_spec((10 * D, HID)),
        _full_spec((1, HID)),
    ],
    out_specs=[
        pl.BlockSpec((RB, HID), lambda i: (i, 0)),
        _full_spec((8, HID)),
    ],
    out_shape=[
        jax.ShapeDtypeStruct((N, HID), jnp.float32),
        jax.ShapeDtypeStruct((8, HID), jnp.float32),
    ],
)

_layer_c = pl.pallas_call(
    _layer_c_body,
    grid=(NB,),
    in_specs=[
        pl.BlockSpec((RB, D), lambda i: (i, 0)),
        pl.BlockSpec((RB, HID), lambda i: (i, 0)),
        _full_spec((8, HID)),
    ],
    out_specs=pl.BlockSpec((RB, D), lambda i: (i, 0)),
    out_shape=jax.ShapeDtypeStruct((N, D), jnp.float32),
)

_readout = pl.pallas_call(
    _readout_body,
    grid=(NB,),
    in_specs=[
        pl.BlockSpec((RB, D), lambda i: (i, 0)),
        _full_spec((D, HID // 2)),
        _full_spec((1, HID // 2)),
        _full_spec((HID // 2, HID // 4)),
        _full_spec((1, HID // 4)),
        _full_spec((HID // 4, 16)),
        _full_spec((1, 16)),
    ],
    out_specs=_full_spec((8, 16)),
    out_shape=jax.ShapeDtypeStruct((8, 16), jnp.float32),
    scratch_shapes=[pltpu.VMEM((8, HID), jnp.float32)],
)


def kernel(edge_index, h, e, snorm_n, W_post, b_post, gamma, beta,
           Wr0, br0, Wr1, br1, Wr2, br2):
    src = edge_index[0].astype(jnp.int32)
    dst = edge_index[1].astype(jnp.int32)

    order = jnp.argsort(dst)
    dst_s = dst[order]
    src_s = src[order]
    row_ptr = jnp.searchsorted(dst_s, jnp.arange(RPLEN, dtype=jnp.int32),
                               side="left").astype(jnp.int32)
    src_p = jnp.concatenate([src_s, jnp.zeros((EPAD - E,), jnp.int32)])
    dst_p = jnp.concatenate([dst_s, jnp.zeros((EPAD - E,), jnp.int32)])

    deg_raw = (row_ptr[1:N + 1] - row_ptr[:N]).astype(jnp.float32)
    deg = jnp.maximum(deg_raw, 1.0)
    log_deg = jnp.log(deg + 1.0)
    scal = jnp.zeros((N, 8), jnp.float32)
    scal = scal.at[:, 0].set(1.0 / deg)
    scal = scal.at[:, 1].set(log_deg / AVG_D_LOG)
    scal = scal.at[:, 2].set(AVG_D_LOG / log_deg)
    scal = scal.at[:, 3].set(snorm_n[:, 0])

    sc_agg = _make_sc_agg()
    b2d = b_post.reshape(L, 1, HID)

    for l in range(L):
        s_a, x_a, n_a = sc_agg(h, src_p, dst_p, row_ptr)
        p, stats = _layer_a(h, s_a[:N], x_a[:N], n_a[:N], scal,
                            W_post[l], b2d[l])
        mu = stats[0] * (1.0 / N)
        var = stats[1] * (1.0 / N) - mu * mu
        a = gamma[l] / jnp.sqrt(var + 1e-5)
        bb = beta[l] - mu * a
        ab = jnp.concatenate([a.reshape(1, HID), bb.reshape(1, HID),
                              jnp.zeros((6, HID), jnp.float32)], axis=0)
        h = _layer_c(h, p, ab)

    w2p = jnp.zeros((HID // 4, 16), jnp.float32).at[:, :OUT].set(Wr2)
    b2p = jnp.zeros((1, 16), jnp.float32).at[0, :OUT].set(br2)
    outp = _readout(h, Wr0, br0.reshape(1, -1), Wr1, br1.reshape(1, -1),
                    w2p, b2p)
    return outp[0, :OUT]


# speedup vs baseline: 5.3882x; 1.5484x over previous
import functools

import jax
import jax.numpy as jnp
from jax import lax
from jax.experimental import pallas as pl
from jax.experimental.pallas import tpu as pltpu
from jax.experimental.pallas import tpu_sc as plsc

N = 10000
E = 320000
D = 128
HID = 128
OUT = 10
L = 4
AVG_D_LOG = 3.4657

NC = 2
NS = 16
NW = NC * NS
NPW = 320
PADN = NW * NPW
CH = 160
NCHUNK = NPW // CH
EB = 128
EPAD = E + EB
RPLEN = PADN + 16

_NEG = float("-inf")
_POS = float("inf")


def _sc_agg_body(h_hbm, src_hbm, dst_hbm, rp_hbm,
                 sum_hbm, max_hbm, min_hbm,
                 rp_v, idx_v, dst_v, msg_v, acc_s, acc_x, acc_n, sem):
    wid = lax.axis_index("s") * NC + lax.axis_index("c")
    base = wid * NPW
    pltpu.sync_copy(rp_hbm.at[pl.ds(base, NPW + 16)], rp_v)

    for c in range(NCHUNK):
        n0 = c * CH
        cabs = base + n0

        def init_body(i, _):
            for r in range(8):
                sl = pl.ds(r * 16, 16)
                acc_s[i, sl] = jnp.zeros((16,), jnp.float32)
                acc_x[i, sl] = jnp.full((16,), _NEG, jnp.float32)
                acc_n[i, sl] = jnp.full((16,), _POS, jnp.float32)
            return 0

        lax.fori_loop(0, CH, init_body, 0)

        e_lo = rp_v[pl.ds(n0, 16)][0]
        e_hi = rp_v[pl.ds(n0 + CH, 16)][0]
        klo = e_lo // EB
        khi = (e_hi + EB - 1) // EB

        def blk_body(kk, _):
            blk = pl.multiple_of((klo + kk) * EB, 8)
            pltpu.sync_copy(src_hbm.at[pl.ds(blk, EB)], idx_v)
            pltpu.sync_copy(dst_hbm.at[pl.ds(blk, EB)], dst_v)
            pltpu.async_copy(h_hbm.at[idx_v], msg_v, sem).wait()

            first_n = dst_v[pl.ds(0, 16)][0]
            last_n = dst_v[pl.ds(EB - 16, 16)][15]
            lo_n = jnp.maximum(first_n, cabs)
            hi_n = jnp.minimum(last_n + 1, cabs + CH)

            def run_body(rn, _):
                node = lo_n + rn
                ln = node - cabs
                v = rp_v[pl.ds(node - base, 16)]
                le = jnp.maximum(v[0], blk) - blk
                he = jnp.maximum(jnp.minimum(v[1], blk + EB) - blk, le)

                regs = []
                for r in range(8):
                    regs.append(acc_s[ln, pl.ds(r * 16, 16)])
                for r in range(8):
                    regs.append(acc_x[ln, pl.ds(r * 16, 16)])
                for r in range(8):
                    regs.append(acc_n[ln, pl.ds(r * 16, 16)])

                def edge_body(jj, rg):
                    j = le + jj
                    out = []
                    for r in range(8):
                        m = msg_v[j, pl.ds(r * 16, 16)]
                        out.append(rg[r] + m)
                    for r in range(8):
                        m = msg_v[j, pl.ds(r * 16, 16)]
                        out.append(jnp.maximum(rg[8 + r], m))
                    for r in range(8):
                        m = msg_v[j, pl.ds(r * 16, 16)]
                        out.append(jnp.minimum(rg[16 + r], m))
                    return tuple(out)

                regs = lax.fori_loop(0, he - le, edge_body, tuple(regs))
                for r in range(8):
                    sl = pl.ds(r * 16, 16)
                    acc_s[ln, sl] = regs[r]
                    acc_x[ln, sl] = regs[8 + r]
                    acc_n[ln, sl] = regs[16 + r]
                return 0

            lax.fori_loop(0, hi_n - lo_n, run_body, 0)
            return 0

        lax.fori_loop(0, khi - klo, blk_body, 0)

        row0 = cabs
        pltpu.sync_copy(acc_s, sum_hbm.at[pl.ds(row0, CH), :])
        pltpu.sync_copy(acc_x, max_hbm.at[pl.ds(row0, CH), :])
        pltpu.sync_copy(acc_n, min_hbm.at[pl.ds(row0, CH), :])


def _make_sc_agg():
    mesh = plsc.VectorSubcoreMesh(core_axis_name="c", subcore_axis_name="s")
    return pl.kernel(
        _sc_agg_body,
        out_type=[jax.ShapeDtypeStruct((PADN, D), jnp.float32)] * 3,
        mesh=mesh,
        scratch_types=[
            pltpu.VMEM((NPW + 16,), jnp.int32),
            pltpu.VMEM((EB,), jnp.int32),
            pltpu.VMEM((EB,), jnp.int32),
            pltpu.VMEM((EB, D), jnp.float32),
            pltpu.VMEM((CH, D), jnp.float32),
            pltpu.VMEM((CH, D), jnp.float32),
            pltpu.VMEM((CH, D), jnp.float32),
            pltpu.SemaphoreType.DMA,
        ],
    )


RB = 400
NB = N // RB


def _layer_a_body(h_ref, s_ref, x_ref, n_ref, scal_ref, w_ref, b_ref,
                  p_ref, stats_ref):
    i = pl.program_id(0)
    inv_deg = scal_ref[:, 0:1]
    s_amp = scal_ref[:, 1:2]
    s_att = scal_ref[:, 2:3]
    snorm = scal_ref[:, 3:4]
    mean = s_ref[...] * inv_deg
    mx = x_ref[...]
    mx = jnp.where(jnp.isfinite(mx), mx, 0.0)
    mn = n_ref[...]
    mn = jnp.where(jnp.isfinite(mn), mn, 0.0)
    h = h_ref[...]
    hcat = jnp.concatenate(
        [h, mean, mx, mn,
         mean * s_amp, mx * s_amp, mn * s_amp,
         mean * s_att, mx * s_att, mn * s_att], axis=1)
    p = jnp.dot(hcat, w_ref[...], preferred_element_type=jnp.float32)
    p = p + b_ref[0:1, :]
    p = jnp.maximum(p, 0.0) * snorm
    p_ref[...] = p

    @pl.when(i == 0)
    def _():
        stats_ref[...] = jnp.zeros_like(stats_ref)

    stats_ref[0:1, :] += jnp.sum(p, axis=0, keepdims=True)
    stats_ref[1:2, :] += jnp.sum(p * p, axis=0, keepdims=True)


def _layer_c_body(h_ref, p_ref, ab_ref, o_ref):
    o_ref[...] = h_ref[...] + p_ref[...] * ab_ref[0:1, :] + ab_ref[1:2, :]


def _readout_body(h_ref, w0_ref, b0_ref, w1_ref, b1_ref, w2_ref, b2_ref,
                  o_ref, acc_ref):
    i = pl.program_id(0)

    @pl.when(i == 0)
    def _():
        acc_ref[...] = jnp.zeros_like(acc_ref)

    acc_ref[0:1, :] += jnp.sum(h_ref[...], axis=0, keepdims=True)

    @pl.when(i == NB - 1)
    def _():
        hg = acc_ref[0:1, :] * (1.0 / N)
        x = jnp.maximum(jnp.dot(hg, w0_ref[...],
                                preferred_element_type=jnp.float32)
                        + b0_ref[0:1, :], 0.0)
        x = jnp.maximum(jnp.dot(x, w1_ref[...],
                                preferred_element_type=jnp.float32)
                        + b1_ref[0:1, :], 0.0)
        x = jnp.dot(x, w2_ref[...], preferred_element_type=jnp.float32) \
            + b2_ref[0:1, :]
        o_ref[...] = jnp.broadcast_to(x, o_ref.shape)


def _full_spec(shape):
    return pl.BlockSpec(shape, lambda i: (0,) * len(shape))


_layer_a = pl.pallas_call(
    _layer_a_body,
    grid=(NB,),
    in_specs=[
        pl.BlockSpec((RB, D), lambda i: (i, 0)),
        pl.BlockSpec((RB, D), lambda i: (i, 0)),
        pl.BlockSpec((RB, D), lambda i: (i, 0)),
        pl.BlockSpec((RB, D), lambda i: (i, 0)),
        pl.BlockSpec((RB, 8), lambda i: (i, 0)),
        _full_spec((10 * D, HID)),
        _full_spec((1, HID)),
    ],
    out_specs=[
        pl.BlockSpec((RB, HID), lambda i: (i, 0)),
        _full_spec((8, HID)),
    ],
    out_shape=[
        jax.ShapeDtypeStruct((N, HID), jnp.float32),
        jax.ShapeDtypeStruct((8, HID), jnp.float32),
    ],
)

_layer_c = pl.pallas_call(
    _layer_c_body,
    grid=(NB,),
    in_specs=[
        pl.BlockSpec((RB, D), lambda i: (i, 0)),
        pl.BlockSpec((RB, HID), lambda i: (i, 0)),
        _full_spec((8, HID)),
    ],
    out_specs=pl.BlockSpec((RB, D), lambda i: (i, 0)),
    out_shape=jax.ShapeDtypeStruct((N, D), jnp.float32),
)

_readout = pl.pallas_call(
    _readout_body,
    grid=(NB,),
    in_specs=[
        pl.BlockSpec((RB, D), lambda i: (i, 0)),
        _full_spec((D, HID // 2)),
        _full_spec((1, HID // 2)),
        _full_spec((HID // 2, HID // 4)),
        _full_spec((1, HID // 4)),
        _full_spec((HID // 4, 16)),
        _full_spec((1, 16)),
    ],
    out_specs=_full_spec((8, 16)),
    out_shape=jax.ShapeDtypeStruct((8, 16), jnp.float32),
    scratch_shapes=[pltpu.VMEM((8, HID), jnp.float32)],
)


def kernel(edge_index, h, e, snorm_n, W_post, b_post, gamma, beta,
           Wr0, br0, Wr1, br1, Wr2, br2):
    src = edge_index[0].astype(jnp.int32)
    dst = edge_index[1].astype(jnp.int32)

    order = jnp.argsort(dst)
    dst_s = dst[order]
    src_s = src[order]
    row_ptr = jnp.searchsorted(dst_s, jnp.arange(RPLEN, dtype=jnp.int32),
                               side="left").astype(jnp.int32)
    src_p = jnp.concatenate([src_s, jnp.zeros((EPAD - E,), jnp.int32)])
    dst_p = jnp.concatenate([dst_s, jnp.zeros((EPAD - E,), jnp.int32)])

    deg_raw = (row_ptr[1:N + 1] - row_ptr[:N]).astype(jnp.float32)
    deg = jnp.maximum(deg_raw, 1.0)
    log_deg = jnp.log(deg + 1.0)
    scal = jnp.zeros((N, 8), jnp.float32)
    scal = scal.at[:, 0].set(1.0 / deg)
    scal = scal.at[:, 1].set(log_deg / AVG_D_LOG)
    scal = scal.at[:, 2].set(AVG_D_LOG / log_deg)
    scal = scal.at[:, 3].set(snorm_n[:, 0])

    sc_agg = _make_sc_agg()
    b2d = b_post.reshape(L, 1, HID)

    for l in range(L):
        s_a, x_a, n_a = sc_agg(h, src_p, dst_p, row_ptr)
        p, stats = _layer_a(h, s_a[:N], x_a[:N], n_a[:N], scal,
                            W_post[l], b2d[l])
        mu = stats[0] * (1.0 / N)
        var = stats[1] * (1.0 / N) - mu * mu
        a = gamma[l] / jnp.sqrt(var + 1e-5)
        bb = beta[l] - mu * a
        ab = jnp.concatenate([a.reshape(1, HID), bb.reshape(1, HID),
                              jnp.zeros((6, HID), jnp.float32)], axis=0)
        h = _layer_c(h, p, ab)

    w2p = jnp.zeros((HID // 4, 16), jnp.float32).at[:, :OUT].set(Wr2)
    b2p = jnp.zeros((1, 16), jnp.float32).at[0, :OUT].set(br2)
    outp = _readout(h, Wr0, br0.reshape(1, -1), Wr1, br1.reshape(1, -1),
                    w2p, b2p)
    return outp[0, :OUT]
